# Initial kernel scaffold; baseline (speedup 1.0000x reference)
import jax
import jax.numpy as jnp
from jax.experimental import pallas as pl


def kernel(concat_embeddings, position_table):
    raise NotImplementedError("write your pallas kernel here")


# speedup vs baseline: 3.2142x; 3.2142x over previous
import jax
import jax.numpy as jnp
from jax.experimental import pallas as pl


def _add_body(concat_ref, table_ref, out_ref):
    out_ref[...] = concat_ref[...] + table_ref[...][None, :, :]


def kernel(concat_embeddings, position_table):
    batch, seq, hidden = concat_embeddings.shape
    bs = 512
    grid = (seq // bs,)
    table = position_table[:seq]
    return pl.pallas_call(
        _add_body,
        grid=grid,
        in_specs=[
            pl.BlockSpec((batch, bs, hidden), lambda i: (0, i, 0)),
            pl.BlockSpec((bs, hidden), lambda i: (i, 0)),
        ],
        out_specs=pl.BlockSpec((batch, bs, hidden), lambda i: (0, i, 0)),
        out_shape=jax.ShapeDtypeStruct((batch, seq, hidden), concat_embeddings.dtype),
    )(concat_embeddings, table)
